# baseline (speedup 1.0000x reference)
import jax
import jax.numpy as jnp
from jax.experimental import pallas as pl
from jax.experimental.pallas import tpu as pltpu
from functools import partial

N_NODES = 10000
BLK = 5000
GRID = N_NODES // BLK
NUM_GRAPHS = 64


def _fused_kernel(batch_ref, x_ref, u_ref, wg_ref, bg_ref, wu_ref, bu_ref,
                  out_ref, acc_ref, cnt_ref, c_ref):
    i = pl.program_id(0)
    f_x = x_ref.shape[1]

    @pl.when(i == 0)
    def _init():
        acc_ref[...] = jnp.zeros_like(acc_ref)
        cnt_ref[...] = jnp.zeros_like(cnt_ref)
        c_ref[...] = (jnp.dot(u_ref[...].astype(jnp.bfloat16),
                              wg_ref[f_x:, :].astype(jnp.bfloat16),
                              preferred_element_type=jnp.float32)
                      + bg_ref[...]).astype(jnp.bfloat16)

    ids = batch_ref[0, 0, :]
    seg = jax.lax.broadcasted_iota(jnp.int32, (NUM_GRAPHS, BLK), 0)
    ohT = (ids[None, :] == seg).astype(jnp.bfloat16)

    x = x_ref[...]
    a = (jnp.dot(x.astype(jnp.bfloat16), wg_ref[:f_x, :].astype(jnp.bfloat16),
                 preferred_element_type=jnp.float32)
         + jax.lax.dot_general(ohT, c_ref[...], (((0,), (0,)), ((), ())),
                               preferred_element_type=jnp.float32))
    w = (x * a).astype(jnp.bfloat16)

    acc_ref[...] += jnp.dot(ohT, w,
                            preferred_element_type=jnp.float32)
    cnt_ref[...] += jnp.dot(ohT, jnp.ones((BLK, 128), jnp.bfloat16),
                            preferred_element_type=jnp.float32)

    @pl.when(i == GRID - 1)
    def _finish():
        x_agg = acc_ref[...] / jnp.maximum(cnt_ref[:, :1], 1.0)
        out_ref[...] = (
            jnp.dot(x_agg, wu_ref[:f_x, :],
                    preferred_element_type=jnp.float32)
            + jnp.dot(u_ref[...], wu_ref[f_x:, :],
                      preferred_element_type=jnp.float32)
            + bu_ref[...])


@partial(jax.jit, static_argnames=())
def kernel(x, edge_index, edge_attr, u, batch, W_g, b_g, W_u, b_u):
    del edge_index, edge_attr
    f_x = x.shape[1]
    f_out = W_u.shape[1]
    batch3 = batch.astype(jnp.int32).reshape(GRID, 1, BLK)

    return pl.pallas_call(
        _fused_kernel,
        grid=(GRID,),
        in_specs=[
            pl.BlockSpec((1, 1, BLK), lambda i: (i, 0, 0)),
            pl.BlockSpec((BLK, f_x), lambda i: (i, 0)),
            pl.BlockSpec(u.shape, lambda i: (0, 0)),
            pl.BlockSpec(W_g.shape, lambda i: (0, 0)),
            pl.BlockSpec(b_g.shape, lambda i: (0,)),
            pl.BlockSpec(W_u.shape, lambda i: (0, 0)),
            pl.BlockSpec(b_u.shape, lambda i: (0,)),
        ],
        out_specs=pl.BlockSpec((NUM_GRAPHS, f_out), lambda i: (0, 0)),
        out_shape=jax.ShapeDtypeStruct((NUM_GRAPHS, f_out), jnp.float32),
        scratch_shapes=[
            pltpu.VMEM((NUM_GRAPHS, f_x), jnp.float32),
            pltpu.VMEM((NUM_GRAPHS, 128), jnp.float32),
            pltpu.VMEM((NUM_GRAPHS, f_x), jnp.bfloat16),
        ],
    )(batch3, x, u, W_g, b_g, W_u, b_u)


# speedup vs baseline: 1.1325x; 1.1325x over previous
import jax
import jax.numpy as jnp
from jax.experimental import pallas as pl
from jax.experimental.pallas import tpu as pltpu
from functools import partial

N_NODES = 10000
BLK = 5000
GRID = N_NODES // BLK
NUM_GRAPHS = 64


def _fused_kernel(batch_ref, x_ref, u_ref, wg_ref, bg_ref, wu_ref, bu_ref,
                  out_ref, acc_ref, cnt_ref, c_ref, oh_ref):
    i = pl.program_id(0)
    f_x = x_ref.shape[1]

    @pl.when(i == 0)
    def _init():
        acc_ref[...] = jnp.zeros_like(acc_ref)
        c_ref[...] = (jnp.dot(u_ref[...].astype(jnp.bfloat16),
                              wg_ref[f_x:, :].astype(jnp.bfloat16),
                              preferred_element_type=jnp.float32)
                      + bg_ref[...]).astype(jnp.bfloat16)
        ids = batch_ref[...]
        seg = jax.lax.broadcasted_iota(jnp.int32, (N_NODES, NUM_GRAPHS), 1)
        oh = (ids[:, None] == seg).astype(jnp.float32)
        oh_ref[...] = oh
        cnt_ref[...] = jnp.broadcast_to(
            jnp.sum(oh, axis=0)[:, None], cnt_ref.shape)

    onehot = oh_ref[pl.ds(i * BLK, BLK), :].astype(jnp.bfloat16)
    x = x_ref[...]
    a = (jnp.dot(x.astype(jnp.bfloat16), wg_ref[:f_x, :].astype(jnp.bfloat16),
                 preferred_element_type=jnp.float32)
         + jnp.dot(onehot, c_ref[...],
                   preferred_element_type=jnp.float32))
    w = (x * a).astype(jnp.bfloat16)

    acc_ref[...] += jax.lax.dot_general(
        onehot, w, (((0,), (0,)), ((), ())),
        preferred_element_type=jnp.float32)

    @pl.when(i == GRID - 1)
    def _finish():
        x_agg = acc_ref[...] / jnp.maximum(cnt_ref[:, :1], 1.0)
        out_ref[...] = (
            jnp.dot(x_agg, wu_ref[:f_x, :],
                    preferred_element_type=jnp.float32)
            + jnp.dot(u_ref[...], wu_ref[f_x:, :],
                      preferred_element_type=jnp.float32)
            + bu_ref[...])


@partial(jax.jit, static_argnames=())
def kernel(x, edge_index, edge_attr, u, batch, W_g, b_g, W_u, b_u):
    del edge_index, edge_attr
    f_x = x.shape[1]
    f_out = W_u.shape[1]

    return pl.pallas_call(
        _fused_kernel,
        grid=(GRID,),
        in_specs=[
            pl.BlockSpec((N_NODES,), lambda i: (0,)),
            pl.BlockSpec((BLK, f_x), lambda i: (i, 0)),
            pl.BlockSpec(u.shape, lambda i: (0, 0)),
            pl.BlockSpec(W_g.shape, lambda i: (0, 0)),
            pl.BlockSpec(b_g.shape, lambda i: (0,)),
            pl.BlockSpec(W_u.shape, lambda i: (0, 0)),
            pl.BlockSpec(b_u.shape, lambda i: (0,)),
        ],
        out_specs=pl.BlockSpec((NUM_GRAPHS, f_out), lambda i: (0, 0)),
        out_shape=jax.ShapeDtypeStruct((NUM_GRAPHS, f_out), jnp.float32),
        scratch_shapes=[
            pltpu.VMEM((NUM_GRAPHS, f_x), jnp.float32),
            pltpu.VMEM((NUM_GRAPHS, 128), jnp.float32),
            pltpu.VMEM((NUM_GRAPHS, f_x), jnp.bfloat16),
            pltpu.VMEM((N_NODES, NUM_GRAPHS), jnp.float32),
        ],
    )(batch.astype(jnp.int32), x, u, W_g, b_g, W_u, b_u)
